# baseline (speedup 1.0000x reference)
import functools

import jax
import jax.numpy as jnp
from jax import lax
from jax.experimental import pallas as pl
from jax.experimental.pallas import tpu as pltpu
from jax.experimental.pallas import tpu_sc as plsc

_N = 32768
_M = 2048
_K = 16
_D = 128
_INV_GRID = 50.0

_NR = 2048


def _bf(a):
    return a.astype(jnp.bfloat16)


def _dense_body(xp_ref, pp_ref, w1_ref, b1_ref, w2f_ref, w2r_ref, u_ref):
    f = jnp.maximum(jnp.dot(_bf(xp_ref[...]), _bf(w1_ref[...]),
                            preferred_element_type=jnp.float32) + b1_ref[...], 0.0)
    pg = _bf(pp_ref[...] * _INV_GRID)
    u_ref[...] = (jnp.dot(_bf(f), _bf(w2f_ref[...]),
                          preferred_element_type=jnp.float32)
                  + jnp.dot(pg, _bf(w2r_ref[...]),
                            preferred_element_type=jnp.float32))


def _dense(xp, pp, w1p, b1r, w2f, w2rg):
    return pl.pallas_call(
        _dense_body,
        grid=(_N // _NR,),
        in_specs=[
            pl.BlockSpec((_NR, 8), lambda i: (i, 0)),
            pl.BlockSpec((_NR, 8), lambda i: (i, 0)),
            pl.BlockSpec((8, _D), lambda i: (0, 0)),
            pl.BlockSpec((1, _D), lambda i: (0, 0)),
            pl.BlockSpec((_D, _D), lambda i: (0, 0)),
            pl.BlockSpec((8, _D), lambda i: (0, 0)),
        ],
        out_specs=pl.BlockSpec((_NR, _D), lambda i: (i, 0)),
        out_shape=jax.ShapeDtypeStruct((_N, _D), jnp.float32),
    )(xp, pp, w1p, b1r, w2f, w2rg)


_MQ = 64
_NF = 32
_FC = _N // _NF


def _pn_body(pt_ref, pn_ref):
    p0 = pt_ref[0:1, :]
    p1 = pt_ref[1:2, :]
    p2 = pt_ref[2:3, :]
    pn_ref[...] = p0 * p0 + p1 * p1 + p2 * p2


def _pn(pt8):
    return pl.pallas_call(
        _pn_body,
        grid=(1,),
        in_specs=[pl.BlockSpec((8, _N), lambda i: (0, 0))],
        out_specs=pl.BlockSpec((1, _N), lambda i: (0, 0)),
        out_shape=jax.ShapeDtypeStruct((1, _N), jnp.float32),
    )(pt8)


def _knn_body(q_ref, pt_ref, pn_ref, idx_ref, d_ref):
    pn = pn_ref[...]
    q0 = q_ref[:, 0:1]
    q1 = q_ref[:, 1:2]
    q2 = q_ref[:, 2:3]
    qn = q0 * q0 + q1 * q1 + q2 * q2
    qb = _bf(q_ref[...])
    ptb = pt_ref[...]

    inf = jnp.float32(jnp.inf)
    m1 = jnp.full((_MQ, _FC), inf)
    f1 = jnp.zeros((_MQ, _FC), jnp.int32)
    m2 = jnp.full((_MQ, _FC), inf)
    f2 = jnp.zeros((_MQ, _FC), jnp.int32)
    s = jnp.dot(qb, ptb, preferred_element_type=jnp.float32)
    for k in range(_NF):
        sl = (qn - 2.0 * s[:, k * _FC : (k + 1) * _FC]) \
            + pn[:, k * _FC : (k + 1) * _FC]
        d_ref[:, k * _FC : (k + 1) * _FC] = sl
        c1 = sl < m1
        c2 = sl < m2
        m2 = jnp.minimum(m2, jnp.maximum(m1, sl))
        f2 = jnp.where(c1, f1, jnp.where(c2, jnp.int32(k), f2))
        m1 = jnp.minimum(m1, sl)
        f1 = jnp.where(c1, jnp.int32(k), f1)
    colio = lax.broadcasted_iota(jnp.int32, (_MQ, _FC), 1)
    g0 = jnp.concatenate([m1, m2], axis=1)
    gidx = jnp.concatenate([f1 * _FC + colio, f2 * _FC + colio], axis=1)
    g = g0
    v16 = None
    for t in range(_K):
        v16 = jnp.min(g, axis=1, keepdims=True)
        g = jnp.where(g == v16, inf, g)
    ci = jnp.where(g0 <= v16, gidx, jnp.int32(_N))
    for t in range(_K):
        mm = jnp.min(ci, axis=1, keepdims=True)
        idx_ref[:, t : t + 1] = mm
        ci = jnp.where(ci == mm, jnp.int32(_N), ci)
    cnt = jnp.sum(jnp.where(d_ref[...] <= v16, 1.0, 0.0), axis=1,
                  keepdims=True)
    mism = jnp.sum(jnp.abs(cnt - jnp.float32(_K)))

    @pl.when(mism > 0.0)
    def _fallback():
        it = lax.broadcasted_iota(jnp.int32, (_MQ, _N), 1)
        for t in range(_K):
            dd = d_ref[...]
            mm = jnp.min(dd, axis=1, keepdims=True)
            cc = jnp.where(dd == mm, it, jnp.int32(_N))
            am = jnp.min(cc, axis=1, keepdims=True)
            idx_ref[:, t : t + 1] = am
            d_ref[...] = jnp.where(it == am, inf, dd)


def _knn(qp, ptb, pnr):
    return pl.pallas_call(
        _knn_body,
        grid=(_M // _MQ,),
        in_specs=[
            pl.BlockSpec((_MQ, 8), lambda i: (i, 0)),
            pl.BlockSpec((8, _N), lambda i: (0, 0)),
            pl.BlockSpec((1, _N), lambda i: (0, 0)),
        ],
        out_specs=pl.BlockSpec((_MQ, _K), lambda i: (i, 0)),
        out_shape=jax.ShapeDtypeStruct((_M, _K), jnp.int32),
        scratch_shapes=[pltpu.VMEM((_MQ, _N), jnp.float32)],
    )(qp, ptb, pnr)


_NC = 2
_NS = 16
_NW = _NC * _NS
_QW = _M // _NW
_QC = 8
_CH = _QW // _QC
_RC = _QC * _K


def _sc_gather_max(u, idx_flat):
    mesh = plsc.VectorSubcoreMesh(core_axis_name="c", subcore_axis_name="s")

    @functools.partial(
        pl.kernel,
        mesh=mesh,
        out_type=jax.ShapeDtypeStruct((_M, _D), jnp.float32),
        scratch_types=[
            pltpu.VMEM((_RC,), jnp.int32),
            pltpu.VMEM((_RC, _D), jnp.float32),
            pltpu.VMEM((_QC, _D), jnp.float32),
            pltpu.SemaphoreType.DMA,
        ],
    )
    def k(u_hbm, idx_hbm, out_hbm, idx_v, rows_v, mx_v, sem):
        wid = lax.axis_index("s") * _NC + lax.axis_index("c")

        def chunk_body(ch, carry):
            qbase = wid * _QW + ch * _QC
            rbase = qbase * _K
            pltpu.sync_copy(idx_hbm.at[pl.ds(rbase, _RC)], idx_v)
            pltpu.async_copy(u_hbm.at[idx_v], rows_v, sem).wait()

            def q_body(q, c2):
                for c in range(_D // 16):
                    acc = rows_v[q * _K, pl.ds(c * 16, 16)]
                    for j in range(1, _K):
                        acc = jnp.maximum(acc, rows_v[q * _K + j, pl.ds(c * 16, 16)])
                    mx_v[q, pl.ds(c * 16, 16)] = acc
                return c2

            lax.fori_loop(0, _QC, q_body, 0)
            pltpu.sync_copy(mx_v, out_hbm.at[pl.ds(qbase, _QC)])
            return carry

        lax.fori_loop(0, _CH, chunk_body, 0)

    return k(u, idx_flat)


def _combine_body(mx_ref, q_ref, w2r_ref, b2_ref, out_ref):
    v = jnp.dot(_bf(q_ref[...] * _INV_GRID), _bf(w2r_ref[...]),
                preferred_element_type=jnp.float32) - b2_ref[...]
    loc = jnp.maximum(mx_ref[...] - v, 0.0)
    out_ref[...] = (jnp.max(loc, axis=0, keepdims=True)
                    + jnp.mean(loc, axis=0, keepdims=True))


def _combine(mx, qp, w2rg, b2r):
    return pl.pallas_call(
        _combine_body,
        grid=(1,),
        in_specs=[
            pl.BlockSpec((_M, _D), lambda i: (0, 0)),
            pl.BlockSpec((_M, 8), lambda i: (0, 0)),
            pl.BlockSpec((8, _D), lambda i: (0, 0)),
            pl.BlockSpec((1, _D), lambda i: (0, 0)),
        ],
        out_specs=pl.BlockSpec((1, _D), lambda i: (0, 0)),
        out_shape=jax.ShapeDtypeStruct((1, _D), jnp.float32),
    )(mx, qp, w2rg, b2r)


def kernel(x, W1, b1, W2, b2):
    points = x[:, :3]
    xp = jnp.pad(x, ((0, 0), (0, 2)))
    pp = jnp.pad(points, ((0, 0), (0, 5)))
    w1p = jnp.pad(W1, ((0, 2), (0, 0)))
    w2f = W2[:_D]
    w2rg = jnp.pad(W2[_D:], ((0, 5), (0, 0)))
    b1r = b1.reshape(1, _D)
    b2r = b2.reshape(1, _D)
    qp = pp[::16]
    pt8 = jnp.pad(points.T, ((0, 5), (0, 0)))

    u = _dense(xp, pp, w1p, b1r, w2f, w2rg)
    idx = _knn(qp, pt8.astype(jnp.bfloat16), _pn(pt8))
    mx = _sc_gather_max(u, idx.reshape(-1))
    out = _combine(mx, qp, w2rg, b2r)
    return out.reshape(_D)


# speedup vs baseline: 1.0229x; 1.0229x over previous
import functools

import jax
import jax.numpy as jnp
from jax import lax
from jax.experimental import pallas as pl
from jax.experimental.pallas import tpu as pltpu
from jax.experimental.pallas import tpu_sc as plsc

_N = 32768
_M = 2048
_K = 16
_D = 128
_INV_GRID = 50.0

_NR = 2048


def _bf(a):
    return a.astype(jnp.bfloat16)


def _dense_body(xp_ref, pp_ref, w1_ref, b1_ref, w2f_ref, w2r_ref, u_ref):
    f = jnp.maximum(jnp.dot(_bf(xp_ref[...]), _bf(w1_ref[...]),
                            preferred_element_type=jnp.float32) + b1_ref[...], 0.0)
    pg = _bf(pp_ref[...] * _INV_GRID)
    u_ref[...] = (jnp.dot(_bf(f), _bf(w2f_ref[...]),
                          preferred_element_type=jnp.float32)
                  + jnp.dot(pg, _bf(w2r_ref[...]),
                            preferred_element_type=jnp.float32))


def _dense(xp, pp, w1p, b1r, w2f, w2rg):
    return pl.pallas_call(
        _dense_body,
        grid=(_N // _NR,),
        in_specs=[
            pl.BlockSpec((_NR, 8), lambda i: (i, 0)),
            pl.BlockSpec((_NR, 8), lambda i: (i, 0)),
            pl.BlockSpec((8, _D), lambda i: (0, 0)),
            pl.BlockSpec((1, _D), lambda i: (0, 0)),
            pl.BlockSpec((_D, _D), lambda i: (0, 0)),
            pl.BlockSpec((8, _D), lambda i: (0, 0)),
        ],
        out_specs=pl.BlockSpec((_NR, _D), lambda i: (i, 0)),
        out_shape=jax.ShapeDtypeStruct((_N, _D), jnp.float32),
    )(xp, pp, w1p, b1r, w2f, w2rg)


_MQ = 64
_NF = 32
_FC = _N // _NF


def _knn_body(q_ref, pt_ref, idx_ref, d_ref):
    p0 = pt_ref[0:1, :]
    p1 = pt_ref[1:2, :]
    p2 = pt_ref[2:3, :]
    pn = p0 * p0 + p1 * p1 + p2 * p2
    q0 = q_ref[:, 0:1]
    q1 = q_ref[:, 1:2]
    q2 = q_ref[:, 2:3]
    qn = q0 * q0 + q1 * q1 + q2 * q2
    qb = _bf(q_ref[...])
    ptb = _bf(pt_ref[...])

    inf = jnp.float32(jnp.inf)
    m1 = jnp.full((_MQ, _FC), inf)
    f1 = jnp.zeros((_MQ, _FC), jnp.int32)
    m2 = jnp.full((_MQ, _FC), inf)
    f2 = jnp.zeros((_MQ, _FC), jnp.int32)
    s = jnp.dot(qb, ptb, preferred_element_type=jnp.float32)
    for k in range(_NF):
        sl = (qn - 2.0 * s[:, k * _FC : (k + 1) * _FC]) \
            + pn[:, k * _FC : (k + 1) * _FC]
        d_ref[:, k * _FC : (k + 1) * _FC] = sl
        c1 = sl < m1
        c2 = sl < m2
        m2 = jnp.minimum(m2, jnp.maximum(m1, sl))
        f2 = jnp.where(c1, f1, jnp.where(c2, jnp.int32(k), f2))
        m1 = jnp.minimum(m1, sl)
        f1 = jnp.where(c1, jnp.int32(k), f1)
    colio = lax.broadcasted_iota(jnp.int32, (_MQ, _FC), 1)
    g0 = jnp.concatenate([m1, m2], axis=1)
    gidx = jnp.concatenate([f1 * _FC + colio, f2 * _FC + colio], axis=1)
    g = g0
    v16 = None
    for t in range(_K):
        v16 = jnp.min(g, axis=1, keepdims=True)
        g = jnp.where(g == v16, inf, g)
    ci = jnp.where(g0 <= v16, gidx, jnp.int32(_N))
    for t in range(_K):
        mm = jnp.min(ci, axis=1, keepdims=True)
        idx_ref[:, t : t + 1] = mm
        ci = jnp.where(ci == mm, jnp.int32(_N), ci)
    cnt = jnp.sum(jnp.where(d_ref[...] <= v16, 1.0, 0.0), axis=1,
                  keepdims=True)
    mism = jnp.sum(jnp.abs(cnt - jnp.float32(_K)))

    @pl.when(mism > 0.0)
    def _fallback():
        it = lax.broadcasted_iota(jnp.int32, (_MQ, _N), 1)
        for t in range(_K):
            dd = d_ref[...]
            mm = jnp.min(dd, axis=1, keepdims=True)
            cc = jnp.where(dd == mm, it, jnp.int32(_N))
            am = jnp.min(cc, axis=1, keepdims=True)
            idx_ref[:, t : t + 1] = am
            d_ref[...] = jnp.where(it == am, inf, dd)


def _knn(qp, pt8):
    return pl.pallas_call(
        _knn_body,
        grid=(_M // _MQ,),
        in_specs=[
            pl.BlockSpec((_MQ, 8), lambda i: (i, 0)),
            pl.BlockSpec((8, _N), lambda i: (0, 0)),
        ],
        out_specs=pl.BlockSpec((_MQ, _K), lambda i: (i, 0)),
        out_shape=jax.ShapeDtypeStruct((_M, _K), jnp.int32),
        scratch_shapes=[pltpu.VMEM((_MQ, _N), jnp.float32)],
    )(qp, pt8)


_NC = 2
_NS = 16
_NW = _NC * _NS
_QW = _M // _NW
_QC = 8
_CH = _QW // _QC
_RC = _QC * _K


def _sc_gather_max(u, idx_flat):
    mesh = plsc.VectorSubcoreMesh(core_axis_name="c", subcore_axis_name="s")

    @functools.partial(
        pl.kernel,
        mesh=mesh,
        out_type=jax.ShapeDtypeStruct((_M, _D), jnp.float32),
        scratch_types=[
            pltpu.VMEM((_RC,), jnp.int32),
            pltpu.VMEM((_RC, _D), jnp.float32),
            pltpu.VMEM((_QC, _D), jnp.float32),
            pltpu.SemaphoreType.DMA,
        ],
    )
    def k(u_hbm, idx_hbm, out_hbm, idx_v, rows_v, mx_v, sem):
        wid = lax.axis_index("s") * _NC + lax.axis_index("c")

        def chunk_body(ch, carry):
            qbase = wid * _QW + ch * _QC
            rbase = qbase * _K
            pltpu.sync_copy(idx_hbm.at[pl.ds(rbase, _RC)], idx_v)
            pltpu.async_copy(u_hbm.at[idx_v], rows_v, sem).wait()

            def q_body(q, c2):
                for c in range(_D // 16):
                    acc = rows_v[q * _K, pl.ds(c * 16, 16)]
                    for j in range(1, _K):
                        acc = jnp.maximum(acc, rows_v[q * _K + j, pl.ds(c * 16, 16)])
                    mx_v[q, pl.ds(c * 16, 16)] = acc
                return c2

            lax.fori_loop(0, _QC, q_body, 0)
            pltpu.sync_copy(mx_v, out_hbm.at[pl.ds(qbase, _QC)])
            return carry

        lax.fori_loop(0, _CH, chunk_body, 0)

    return k(u, idx_flat)


def _combine_body(mx_ref, q_ref, w2r_ref, b2_ref, out_ref):
    v = jnp.dot(_bf(q_ref[...] * _INV_GRID), _bf(w2r_ref[...]),
                preferred_element_type=jnp.float32) - b2_ref[...]
    loc = jnp.maximum(mx_ref[...] - v, 0.0)
    out_ref[...] = (jnp.max(loc, axis=0, keepdims=True)
                    + jnp.mean(loc, axis=0, keepdims=True))


def _combine(mx, qp, w2rg, b2r):
    return pl.pallas_call(
        _combine_body,
        grid=(1,),
        in_specs=[
            pl.BlockSpec((_M, _D), lambda i: (0, 0)),
            pl.BlockSpec((_M, 8), lambda i: (0, 0)),
            pl.BlockSpec((8, _D), lambda i: (0, 0)),
            pl.BlockSpec((1, _D), lambda i: (0, 0)),
        ],
        out_specs=pl.BlockSpec((1, _D), lambda i: (0, 0)),
        out_shape=jax.ShapeDtypeStruct((1, _D), jnp.float32),
    )(mx, qp, w2rg, b2r)


def kernel(x, W1, b1, W2, b2):
    points = x[:, :3]
    xp = jnp.pad(x, ((0, 0), (0, 2)))
    pp = jnp.pad(points, ((0, 0), (0, 5)))
    w1p = jnp.pad(W1, ((0, 2), (0, 0)))
    w2f = W2[:_D]
    w2rg = jnp.pad(W2[_D:], ((0, 5), (0, 0)))
    b1r = b1.reshape(1, _D)
    b2r = b2.reshape(1, _D)
    qp = pp[::16]
    pt8 = jnp.pad(points.T, ((0, 5), (0, 0)))

    u = _dense(xp, pp, w1p, b1r, w2f, w2rg)
    idx = _knn(qp, pt8)
    mx = _sc_gather_max(u, idx.reshape(-1))
    out = _combine(mx, qp, w2rg, b2r)
    return out.reshape(_D)
